# baseline (speedup 1.0000x reference)
import dataclasses
import functools

import jax
import jax.numpy as jnp
from jax import lax
from jax.experimental import pallas as pl
from jax.experimental.pallas import tpu as pltpu
from jax.experimental.pallas import tpu_sc as plsc

BLK = 256
PHI_BLK = 512
LANES = 16
NW = 32
SCH = 32


def _sc_compiler_params():
    cp = pltpu.CompilerParams()
    if "needs_layout_passes" in pltpu.CompilerParams.__dataclass_fields__:
        cp = dataclasses.replace(cp, needs_layout_passes=False)
    return cp


def _dot(a, b):
    return jax.lax.dot_general(
        a, b, (((1,), (0,)), ((), ())),
        preferred_element_type=jnp.float32)


def _phi_kernel(x_ref, w_ref, b_ref, xf_ref, pr_ref):
    h = jnp.maximum(_dot(x_ref[...], w_ref[0]) + b_ref[0], 0.0)
    xf = jnp.maximum(_dot(h, w_ref[1]) + b_ref[1], 0.0)
    xf_ref[...] = xf
    pr_ref[...] = jnp.ones_like(pr_ref)


def _phi_half(x, phi_Ws, phi_bs, half):
    B, D = x.shape
    nb = (B // 2) // PHI_BLK
    off = half * nb
    return pl.pallas_call(
        _phi_kernel,
        grid=(nb,),
        in_specs=[
            pl.BlockSpec((PHI_BLK, D), lambda i: (i + off, 0)),
            pl.BlockSpec((2, D, D), lambda i: (0, 0, 0)),
            pl.BlockSpec((2, 1, D), lambda i: (0, 0, 0)),
        ],
        out_specs=[
            pl.BlockSpec((PHI_BLK, D), lambda i: (i, 0)),
            pl.BlockSpec((PHI_BLK, 1), lambda i: (i, 0)),
        ],
        out_shape=[
            jax.ShapeDtypeStruct((B // 2, D), jnp.float32),
            jax.ShapeDtypeStruct((B // 2, 1), jnp.float32),
        ],
        compiler_params=pltpu.CompilerParams(
            dimension_semantics=("arbitrary",)),
    )(x, phi_Ws, phi_bs.reshape(2, 1, D))


def _route(tr):
    B = tr.shape[0]
    B_pad = B + BLK
    nchunks = B // LANES
    mesh = plsc.VectorSubcoreMesh(core_axis_name="c", subcore_axis_name="s")

    @functools.partial(
        pl.kernel,
        out_type=(jax.ShapeDtypeStruct((B,), jnp.int32),
                  jax.ShapeDtypeStruct((B_pad,), jnp.int32),
                  jax.ShapeDtypeStruct((LANES,), jnp.int32)),
        mesh=mesh,
        scratch_types=[pltpu.VMEM((B,), jnp.int32),
                       pltpu.VMEM((B,), jnp.int32),
                       pltpu.VMEM((B_pad,), jnp.int32),
                       pltpu.VMEM((LANES,), jnp.int32),
                       pltpu.SemaphoreType.DMA],
        compiler_params=_sc_compiler_params(),
    )
    def route_kernel(tr_hbm, dest_hbm, half_hbm, info_hbm,
                     tr_v, dest_v, half_v, info_v, sem):
        wid = lax.axis_index("c") * 16 + lax.axis_index("s")

        @pl.when(wid == 0)
        def _():
            pltpu.async_copy(tr_hbm, tr_v, sem).wait()
            ones_v = jnp.full((LANES,), 1, jnp.int32)
            zeros_v = jnp.full((LANES,), 0, jnp.int32)

            def count_zeros(i, acc):
                chunk = tr_v[pl.ds(i * LANES, LANES)]
                return acc + jnp.where(chunk == 0, ones_v, zeros_v)

            acc = lax.fori_loop(0, nchunks, count_zeros, zeros_v)
            n0 = jnp.sum(acc)
            g1 = ((n0 + BLK - 1) // BLK) * BLK

            def fill(i, carry):
                zbase, obase = carry
                chunk = tr_v[pl.ds(i * LANES, LANES)]
                is0 = chunk == 0
                z32 = jnp.where(is0, ones_v, zeros_v)
                zc = plsc.cumsum(z32)
                oc = plsc.cumsum(ones_v - z32)
                dest = jnp.where(is0, zbase + zc - 1, g1 + obase + oc - 1)
                dest_v[pl.ds(i * LANES, LANES)] = dest
                hsel = jnp.where(i >= nchunks // 2, ones_v, zeros_v)
                plsc.store_scatter(half_v, [dest], hsel)
                zsum = jnp.sum(z32)
                return (zbase + zsum, obase + (LANES - zsum))

            lax.fori_loop(0, nchunks, fill, (jnp.int32(0), jnp.int32(0)))

            info_v[...] = jnp.full((LANES,), n0, jnp.int32)
            pltpu.async_copy(dest_v, dest_hbm, sem).wait()
            pltpu.async_copy(half_v, half_hbm, sem).wait()
            pltpu.async_copy(info_v, info_hbm, sem).wait()

    return route_kernel(tr)


def _scatter_half(xh, dest2d, half, B_pad):
    Bh, D = xh.shape
    per_w = Bh // NW
    nch = per_w // SCH
    mesh = plsc.VectorSubcoreMesh(core_axis_name="c", subcore_axis_name="s")

    @functools.partial(
        pl.kernel,
        out_type=jax.ShapeDtypeStruct((B_pad, D), jnp.float32),
        mesh=mesh,
        scratch_types=[pltpu.VMEM((SCH, D), jnp.float32),
                       pltpu.VMEM((SCH, D), jnp.float32),
                       pltpu.VMEM((1, SCH), jnp.int32),
                       pltpu.VMEM((1, SCH), jnp.int32),
                       pltpu.SemaphoreType.DMA,
                       pltpu.SemaphoreType.DMA,
                       pltpu.SemaphoreType.DMA,
                       pltpu.SemaphoreType.DMA],
    )
    def scatter_kernel(xh_hbm, dest_hbm, out_hbm, d0, d1, i0, i1,
                       sl0, sl1, ss0, ss1):
        wid = lax.axis_index("c") * 16 + lax.axis_index("s")
        base = wid * per_w
        rb = (half * Bh + base) // SCH
        l0 = pltpu.async_copy(xh_hbm.at[pl.ds(base, SCH)], d0, sl0)
        l1 = pltpu.async_copy(xh_hbm.at[pl.ds(base + SCH, SCH)], d1, sl1)
        pltpu.sync_copy(dest_hbm.at[pl.ds(rb, 1), :], i0)
        pltpu.sync_copy(dest_hbm.at[pl.ds(rb + 1, 1), :], i1)
        l0.wait()
        s0 = pltpu.async_copy(d0, out_hbm.at[i0.at[0]], ss0)
        l1.wait()
        s1 = pltpu.async_copy(d1, out_hbm.at[i1.at[0]], ss1)
        s0.wait()
        s1.wait()

    return scatter_kernel(xh, dest2d)


def _concat_halves(xa, xb):
    Bh, D = xa.shape
    B = 2 * Bh
    per_w = B // NW
    C = 64
    mesh = plsc.VectorSubcoreMesh(core_axis_name="c", subcore_axis_name="s")

    @functools.partial(
        pl.kernel,
        out_type=jax.ShapeDtypeStruct((B, D), jnp.float32),
        mesh=mesh,
        scratch_types=[pltpu.VMEM((C, D), jnp.float32),
                       pltpu.SemaphoreType.DMA],
    )
    def concat_kernel(xa_hbm, xb_hbm, out_hbm, buf, sem):
        wid = lax.axis_index("c") * 16 + lax.axis_index("s")
        base = wid * per_w

        for c in range(per_w // C):
            g = base + c * C

            @pl.when(g < Bh)
            def _():
                pltpu.async_copy(xa_hbm.at[pl.ds(g, C)], buf, sem).wait()

            @pl.when(g >= Bh)
            def _():
                pltpu.async_copy(xb_hbm.at[pl.ds(g - Bh, C)], buf,
                                 sem).wait()

            pltpu.sync_copy(buf, out_hbm.at[pl.ds(g, C)])

    return concat_kernel(xa, xb)


def _heads_kernel(info_ref, m_ref, xa_ref, xb_ref, w_ref, b_ref, ow_ref,
                  ob_ref, out_ref):
    bidx = pl.program_id(0)
    n0 = info_ref[0]
    nb0 = (n0 + BLK - 1) // BLK
    t = jnp.where(bidx >= nb0, 1, 0)
    start = jnp.where(t == 0, bidx * BLK, n0 + (bidx - nb0) * BLK)
    x = jnp.where(m_ref[...] == 0, xa_ref[...], xb_ref[...])
    for i in range(3):
        x = jnp.maximum(_dot(x, w_ref[t, i]) + b_ref[t, i], 0.0)
    y = jnp.sum(x * ow_ref[t], axis=1, keepdims=True) + ob_ref[t]
    out_ref[pl.ds(start, BLK), :] = y


def _heads(pad_a, pad_b, halfsel, info, yh_Ws, yh_bs, yout_W, yout_b):
    B_pad, D = pad_a.shape
    nb = B_pad // BLK
    return pl.pallas_call(
        _heads_kernel,
        grid=(nb,),
        in_specs=[
            pl.BlockSpec(memory_space=pltpu.SMEM),
            pl.BlockSpec((BLK, 1), lambda i: (i, 0)),
            pl.BlockSpec((BLK, D), lambda i: (i, 0)),
            pl.BlockSpec((BLK, D), lambda i: (i, 0)),
            pl.BlockSpec((2, 3, D, D), lambda i: (0, 0, 0, 0)),
            pl.BlockSpec((2, 3, 1, D), lambda i: (0, 0, 0, 0)),
            pl.BlockSpec((2, 1, D), lambda i: (0, 0, 0)),
            pl.BlockSpec((2, 1, 1), lambda i: (0, 0, 0)),
        ],
        out_specs=pl.BlockSpec((B_pad, 1), lambda i: (0, 0)),
        out_shape=jax.ShapeDtypeStruct((B_pad, 1), jnp.float32),
        compiler_params=pltpu.CompilerParams(
            dimension_semantics=("arbitrary",)),
    )(info, halfsel.reshape(B_pad, 1), pad_a, pad_b, yh_Ws,
      yh_bs.reshape(2, 3, 1, D), yout_W.reshape(2, 1, D),
      yout_b.reshape(2, 1, 1))


def kernel(cofeatures_input, treatment_input, phi_Ws, phi_bs, yh_Ws, yh_bs,
           yout_W, yout_b, t_W, t_b):
    del t_W, t_b
    B, D = cofeatures_input.shape
    B_pad = B + BLK
    tr = treatment_input.reshape(-1).astype(jnp.int32)
    dest, halfsel, info = _route(tr)
    xf_a, pr_a = _phi_half(cofeatures_input, phi_Ws, phi_bs, 0)
    xf_b, pr_b = _phi_half(cofeatures_input, phi_Ws, phi_bs, 1)
    dest2d = dest.reshape(B // SCH, SCH)
    pad_a = _scatter_half(xf_a, dest2d, 0, B_pad)
    pad_b = _scatter_half(xf_b, dest2d, 1, B_pad)
    out_buf = _heads(pad_a, pad_b, halfsel, info, yh_Ws, yh_bs,
                     yout_W, yout_b)
    x_flux = _concat_halves(xf_a, xf_b)
    treatment_probs = jnp.concatenate([pr_a, pr_b], axis=0)
    return (out_buf[:B], x_flux, treatment_probs)


# speedup vs baseline: 1.0814x; 1.0814x over previous
import dataclasses
import functools

import jax
import jax.numpy as jnp
from jax import lax
from jax.experimental import pallas as pl
from jax.experimental.pallas import tpu as pltpu
from jax.experimental.pallas import tpu_sc as plsc

BLK = 256
PHI_BLK = 512
LANES = 16
NW = 32


def _sc_compiler_params():
    cp = pltpu.CompilerParams()
    if "needs_layout_passes" in pltpu.CompilerParams.__dataclass_fields__:
        cp = dataclasses.replace(cp, needs_layout_passes=False)
    return cp


def _dot(a, b):
    return jax.lax.dot_general(
        a, b, (((1,), (0,)), ((), ())),
        preferred_element_type=jnp.float32)


def _bf(x):
    return x.astype(jnp.bfloat16)


def _phi_kernel(x_ref, w_ref, b_ref, xf_ref, pr_ref):
    h = jnp.maximum(_dot(x_ref[...], w_ref[0]) + b_ref[0], 0.0)
    xf = jnp.maximum(_dot(h, w_ref[1]) + b_ref[1], 0.0)
    xf_ref[...] = xf
    pr_ref[...] = jnp.ones_like(pr_ref)


def _phi(x, phi_Ws, phi_bs):
    B, D = x.shape
    nb = B // PHI_BLK
    return pl.pallas_call(
        _phi_kernel,
        grid=(nb,),
        in_specs=[
            pl.BlockSpec((PHI_BLK, D), lambda i: (i, 0)),
            pl.BlockSpec((2, D, D), lambda i: (0, 0, 0)),
            pl.BlockSpec((2, 1, D), lambda i: (0, 0, 0)),
        ],
        out_specs=[
            pl.BlockSpec((PHI_BLK, D), lambda i: (i, 0)),
            pl.BlockSpec((PHI_BLK, 1), lambda i: (i, 0)),
        ],
        out_shape=[
            jax.ShapeDtypeStruct((B, D), jnp.float32),
            jax.ShapeDtypeStruct((B, 1), jnp.float32),
        ],
        compiler_params=pltpu.CompilerParams(
            dimension_semantics=("arbitrary",)),
    )(x, phi_Ws, phi_bs.reshape(2, 1, D))


def _route(tr):
    B = tr.shape[0]
    B_pad = B + BLK
    nchunks = B // LANES
    mesh = plsc.VectorSubcoreMesh(core_axis_name="c", subcore_axis_name="s")

    @functools.partial(
        pl.kernel,
        out_type=(jax.ShapeDtypeStruct((B_pad,), jnp.int32),
                  jax.ShapeDtypeStruct((LANES,), jnp.int32)),
        mesh=mesh,
        scratch_types=[pltpu.VMEM((B,), jnp.int32),
                       pltpu.VMEM((B_pad,), jnp.int32),
                       pltpu.VMEM((LANES,), jnp.int32),
                       pltpu.SemaphoreType.DMA],
        compiler_params=_sc_compiler_params(),
    )
    def route_kernel(tr_hbm, src_hbm, info_hbm, tr_v, src_v, info_v, sem):
        wid = lax.axis_index("c") * 16 + lax.axis_index("s")

        @pl.when(wid == 0)
        def _():
            pltpu.async_copy(tr_hbm, tr_v, sem).wait()
            ones_v = jnp.full((LANES,), 1, jnp.int32)
            zeros_v = jnp.full((LANES,), 0, jnp.int32)

            def count_zeros(i, acc):
                chunk = tr_v[pl.ds(i * LANES, LANES)]
                return acc + jnp.where(chunk == 0, ones_v, zeros_v)

            acc = lax.fori_loop(0, nchunks, count_zeros, zeros_v)
            n0 = jnp.sum(acc)
            g1 = ((n0 + BLK - 1) // BLK) * BLK

            iota_v = lax.iota(jnp.int32, LANES)

            def init(i, c):
                src_v[pl.ds(i * LANES, LANES)] = (
                    (iota_v + i * LANES) & (B - 1))
                return c

            lax.fori_loop(0, B_pad // LANES, init, 0)

            def scatter_dest(i, carry):
                zbase, obase = carry
                chunk = tr_v[pl.ds(i * LANES, LANES)]
                is0 = chunk == 0
                z32 = jnp.where(is0, ones_v, zeros_v)
                zc = plsc.cumsum(z32)
                oc = plsc.cumsum(ones_v - z32)
                dest = jnp.where(is0, zbase + zc - 1, g1 + obase + oc - 1)
                plsc.store_scatter(src_v, [dest], iota_v + i * LANES)
                zsum = jnp.sum(z32)
                return (zbase + zsum, obase + (LANES - zsum))

            lax.fori_loop(0, nchunks, scatter_dest,
                          (jnp.int32(0), jnp.int32(0)))

            info_v[...] = jnp.full((LANES,), n0, jnp.int32)
            pltpu.async_copy(src_v, src_hbm, sem).wait()
            pltpu.async_copy(info_v, info_hbm, sem).wait()

    return route_kernel(tr)


def _gather_rows(xf, src):
    B, D = xf.shape
    B_pad = src.shape[0]
    per_w = B_pad // NW
    CS = (40, 40, 32, per_w - 112)
    CO = (0, 40, 80, 112)
    CB = 40
    mesh = plsc.VectorSubcoreMesh(core_axis_name="c", subcore_axis_name="s")

    @functools.partial(
        pl.kernel,
        out_type=jax.ShapeDtypeStruct((B_pad, D), jnp.float32),
        mesh=mesh,
        scratch_types=[pltpu.VMEM((per_w,), jnp.int32),
                       pltpu.VMEM((CB, D), jnp.float32),
                       pltpu.VMEM((CB, D), jnp.float32),
                       pltpu.VMEM((CB, D), jnp.float32),
                       pltpu.SemaphoreType.DMA,
                       pltpu.SemaphoreType.DMA,
                       pltpu.SemaphoreType.DMA,
                       pltpu.SemaphoreType.DMA,
                       pltpu.SemaphoreType.DMA,
                       pltpu.SemaphoreType.DMA],
    )
    def gather_kernel(xf_hbm, src_hbm, out_hbm, idx_v, buf0, buf1, buf2,
                      sg0, sg1, sg2, sw0, sw1, sw2):
        wid = lax.axis_index("c") * 16 + lax.axis_index("s")
        base = wid * per_w
        bufs = (buf0, buf1, buf2)
        sgs = (sg0, sg1, sg2)
        sws = (sw0, sw1, sw2)
        pltpu.sync_copy(src_hbm.at[pl.ds(base, per_w)], idx_v)
        gathers = {}
        writes = {}
        for k in range(3):
            gathers[k] = pltpu.async_copy(
                xf_hbm.at[idx_v.at[pl.ds(CO[k], CS[k])]],
                bufs[k].at[pl.ds(0, CS[k])], sgs[k])
        gathers[0].wait()
        writes[0] = pltpu.async_copy(
            buf0.at[pl.ds(0, CS[0])], out_hbm.at[pl.ds(base + CO[0], CS[0])],
            sws[0])
        gathers[1].wait()
        writes[1] = pltpu.async_copy(
            buf1.at[pl.ds(0, CS[1])], out_hbm.at[pl.ds(base + CO[1], CS[1])],
            sws[1])
        writes[0].wait()
        gathers[3] = pltpu.async_copy(
            xf_hbm.at[idx_v.at[pl.ds(CO[3], CS[3])]],
            buf0.at[pl.ds(0, CS[3])], sg0)
        gathers[2].wait()
        writes[2] = pltpu.async_copy(
            buf2.at[pl.ds(0, CS[2])], out_hbm.at[pl.ds(base + CO[2], CS[2])],
            sws[2])
        gathers[3].wait()
        writes[3] = pltpu.async_copy(
            buf0.at[pl.ds(0, CS[3])], out_hbm.at[pl.ds(base + CO[3], CS[3])],
            sw0)
        writes[1].wait()
        writes[2].wait()
        writes[3].wait()

    return gather_kernel(xf, src)


def _heads_kernel(info_ref, x_ref, w_ref, b_ref, ow_ref, ob_ref, out_ref):
    bidx = pl.program_id(0)
    n0 = info_ref[0]
    nb0 = (n0 + BLK - 1) // BLK
    t = jnp.where(bidx >= nb0, 1, 0)
    start = jnp.where(t == 0, bidx * BLK, n0 + (bidx - nb0) * BLK)
    x = x_ref[...]
    for i in range(3):
        x = jnp.maximum(_dot(x, w_ref[t, i]) + b_ref[t, i], 0.0)
    y = jnp.sum(x * ow_ref[t], axis=1, keepdims=True) + ob_ref[t]
    out_ref[pl.ds(start, BLK), :] = y


def _heads(padded_x, info, yhW_bf, yh_bs, youtW_bf, yout_b):
    B_pad, D = padded_x.shape
    nb = B_pad // BLK
    return pl.pallas_call(
        _heads_kernel,
        grid=(nb,),
        in_specs=[
            pl.BlockSpec(memory_space=pltpu.SMEM),
            pl.BlockSpec((BLK, D), lambda i: (i, 0)),
            pl.BlockSpec((2, 3, D, D), lambda i: (0, 0, 0, 0)),
            pl.BlockSpec((2, 3, 1, D), lambda i: (0, 0, 0, 0)),
            pl.BlockSpec((2, 1, D), lambda i: (0, 0, 0)),
            pl.BlockSpec((2, 1, 1), lambda i: (0, 0, 0)),
        ],
        out_specs=pl.BlockSpec((B_pad, 1), lambda i: (0, 0)),
        out_shape=jax.ShapeDtypeStruct((B_pad, 1), jnp.float32),
        compiler_params=pltpu.CompilerParams(
            dimension_semantics=("arbitrary",)),
    )(info, padded_x, yhW_bf, yh_bs.reshape(2, 3, 1, D),
      youtW_bf.reshape(2, 1, D), yout_b.reshape(2, 1, 1))


def kernel(cofeatures_input, treatment_input, phi_Ws, phi_bs, yh_Ws, yh_bs,
           yout_W, yout_b, t_W, t_b):
    B, D = cofeatures_input.shape
    tr = treatment_input.reshape(-1).astype(jnp.int32)
    del t_W, t_b
    x_flux, treatment_probs = _phi(cofeatures_input, phi_Ws, phi_bs)
    src, info = _route(tr)
    padded_x = _gather_rows(x_flux, src)
    out_buf = _heads(padded_x, info, yh_Ws, yh_bs, yout_W, yout_b)
    return (out_buf[:B], x_flux, treatment_probs)
